# Initial kernel scaffold; baseline (speedup 1.0000x reference)
import jax
import jax.numpy as jnp
from jax.experimental import pallas as pl


def kernel(x, edge_index, W1, b1, W2, b2):
    raise NotImplementedError("write your pallas kernel here")


# speedup vs baseline: 9.0317x; 9.0317x over previous
import functools

import jax
import jax.numpy as jnp
from jax import lax
from jax.experimental import pallas as pl
from jax.experimental.pallas import tpu as pltpu
from jax.experimental.pallas import tpu_sc as plsc

N = 10000
E = 320000
D = 128

NC = 2
NS = 16
CHUNK = 128
E_PAD = 327680
CHUNKS_TOTAL = E_PAD // CHUNK
CHUNKS_PER_SC = CHUNKS_TOTAL // NC
CHUNKS_PER_TILE = CHUNKS_PER_SC // NS
ACC_ROWS = 10240
ROWS_PER_TILE = ACC_ROWS // NS

_mesh = plsc.VectorSubcoreMesh(core_axis_name="c", subcore_axis_name="s")


@functools.partial(
    pl.kernel,
    out_type=jax.ShapeDtypeStruct((NC, ACC_ROWS), jnp.float32),
    mesh=_mesh,
    scratch_types=[
        pltpu.VMEM((CHUNKS_PER_TILE, CHUNK), jnp.int32),
        pltpu.VMEM((CHUNK,), jnp.float32),
        pltpu.VMEM((ROWS_PER_TILE,), jnp.float32),
        pltpu.VMEM_SHARED((ACC_ROWS,), jnp.float32),
    ],
)
def _deg_kernel(dst_hbm, out_hbm, dst_v, ones_v, zero_v, acc_s):
    c = lax.axis_index("c")
    s = lax.axis_index("s")

    def _fill(i, _):
        ones_v[pl.ds(i * 16, 16)] = jnp.full((16,), 1.0, jnp.float32)
        return ()
    lax.fori_loop(0, CHUNK // 16, _fill, (), unroll=True)

    def _zfill(i, _):
        zero_v[pl.ds(i * 16, 16)] = jnp.zeros((16,), jnp.float32)
        return ()
    lax.fori_loop(0, ROWS_PER_TILE // 16, _zfill, ())

    pltpu.sync_copy(zero_v, acc_s.at[pl.ds(s * ROWS_PER_TILE, ROWS_PER_TILE)])
    plsc.subcore_barrier()

    chunk0 = c * CHUNKS_PER_SC + s * CHUNKS_PER_TILE
    pltpu.sync_copy(dst_hbm.at[pl.ds(chunk0, CHUNKS_PER_TILE)], dst_v)

    def _body(j, _):
        pltpu.sync_copy(ones_v, acc_s.at[dst_v.at[j]], add=True)
        return ()
    lax.fori_loop(0, CHUNKS_PER_TILE, _body, ())

    plsc.subcore_barrier()
    pltpu.sync_copy(
        acc_s.at[pl.ds(s * ROWS_PER_TILE, ROWS_PER_TILE)],
        out_hbm.at[c, pl.ds(s * ROWS_PER_TILE, ROWS_PER_TILE)],
    )


@functools.partial(
    pl.kernel,
    out_type=jax.ShapeDtypeStruct((NC, ACC_ROWS, D), jnp.float32),
    mesh=_mesh,
    scratch_types=[
        pltpu.VMEM((CHUNKS_PER_TILE, CHUNK), jnp.int32),
        pltpu.VMEM((CHUNKS_PER_TILE, CHUNK), jnp.int32),
        pltpu.VMEM((CHUNK, D), jnp.float32),
        pltpu.VMEM((8, D), jnp.float32),
        pltpu.VMEM_SHARED((ACC_ROWS, D), jnp.float32),
        pltpu.SemaphoreType.DMA,
    ],
)
def _agg_kernel(h_hbm, src_hbm, dst_hbm, out_hbm,
                src_v, dst_v, rows_a, zero_v, acc_s, sem_a):
    c = lax.axis_index("c")
    s = lax.axis_index("s")

    def _zfill(i, _):
        def _zl(l, _):
            zero_v[i, pl.ds(l * 16, 16)] = jnp.zeros((16,), jnp.float32)
            return ()
        lax.fori_loop(0, D // 16, _zl, (), unroll=True)
        return ()
    lax.fori_loop(0, 8, _zfill, ())

    row0 = s * ROWS_PER_TILE

    def _zcopy(j, _):
        pltpu.sync_copy(zero_v, acc_s.at[pl.ds(row0 + j * 8, 8)])
        return ()
    lax.fori_loop(0, ROWS_PER_TILE // 8, _zcopy, ())
    plsc.subcore_barrier()

    chunk0 = c * CHUNKS_PER_SC + s * CHUNKS_PER_TILE
    pltpu.sync_copy(src_hbm.at[pl.ds(chunk0, CHUNKS_PER_TILE)], src_v)
    pltpu.sync_copy(dst_hbm.at[pl.ds(chunk0, CHUNKS_PER_TILE)], dst_v)

    def _body(j, _):
        pltpu.async_copy(h_hbm.at[src_v.at[j]], rows_a, sem_a).wait()
        pltpu.sync_copy(rows_a, acc_s.at[dst_v.at[j]], add=True)
        return ()

    lax.fori_loop(0, CHUNKS_PER_TILE, _body, ())

    plsc.subcore_barrier()
    pltpu.sync_copy(
        acc_s.at[pl.ds(row0, ROWS_PER_TILE)],
        out_hbm.at[c, pl.ds(row0, ROWS_PER_TILE)],
    )


_ROW_BLK = 2000
_row_spec = pl.BlockSpec((_ROW_BLK, D), lambda i: (i, 0))
_deg_spec = pl.BlockSpec((_ROW_BLK, 1), lambda i: (i, 0))
_w_spec = pl.BlockSpec((D, D), lambda i: (0, 0))
_b_spec = pl.BlockSpec((1, D), lambda i: (0, 0))


def _tc1_body(x_ref, w_ref, d0_ref, d1_ref, o_ref):
    dis = lax.rsqrt(d0_ref[...] + d1_ref[...] + 1.0)
    h = jnp.dot(x_ref[...], w_ref[...], preferred_element_type=jnp.float32)
    o_ref[...] = h * dis


def _tc2_body(p0_ref, p1_ref, h_ref, d0_ref, d1_ref, w_ref, b_ref, o_ref):
    dis = lax.rsqrt(d0_ref[...] + d1_ref[...] + 1.0)
    z = (p0_ref[...] + p1_ref[...] + h_ref[...]) * dis + b_ref[...]
    r = jnp.maximum(z, 0.0)
    o_ref[...] = jnp.dot(r, w_ref[...], preferred_element_type=jnp.float32) * dis


def _tc3_body(q0_ref, q1_ref, h_ref, d0_ref, d1_ref, b_ref, o_ref):
    dis = lax.rsqrt(d0_ref[...] + d1_ref[...] + 1.0)
    o_ref[...] = (q0_ref[...] + q1_ref[...] + h_ref[...]) * dis + b_ref[...]


_out_nd = jax.ShapeDtypeStruct((N, D), jnp.float32)

_tc1 = pl.pallas_call(
    _tc1_body, grid=(N // _ROW_BLK,),
    in_specs=[_row_spec, _w_spec, _deg_spec, _deg_spec],
    out_specs=_row_spec, out_shape=_out_nd)

_tc2 = pl.pallas_call(
    _tc2_body, grid=(N // _ROW_BLK,),
    in_specs=[_row_spec, _row_spec, _row_spec, _deg_spec, _deg_spec,
              _w_spec, _b_spec],
    out_specs=_row_spec, out_shape=_out_nd)

_tc3 = pl.pallas_call(
    _tc3_body, grid=(N // _ROW_BLK,),
    in_specs=[_row_spec, _row_spec, _row_spec, _deg_spec, _deg_spec, _b_spec],
    out_specs=_row_spec, out_shape=_out_nd)


@jax.jit
def kernel(x, edge_index, W1, b1, W2, b2):
    src = edge_index[0].astype(jnp.int32)
    dst = edge_index[1].astype(jnp.int32)
    pad = E_PAD - E
    src = jnp.concatenate([src, jnp.zeros((pad,), jnp.int32)])
    dst = jnp.concatenate([dst, jnp.full((pad,), N, jnp.int32)])
    src2 = src.reshape(CHUNKS_TOTAL, CHUNK)
    dst2 = dst.reshape(CHUNKS_TOTAL, CHUNK)
    b1r = b1.reshape(1, D)
    b2r = b2.reshape(1, D)

    deg = _deg_kernel(dst2)
    d0 = deg[0, :N].reshape(N, 1)
    d1 = deg[1, :N].reshape(N, 1)

    h1 = _tc1(x, W1, d0, d1)
    p = _agg_kernel(h1, src2, dst2)
    h2 = _tc2(p[0, :N], p[1, :N], h1, d0, d1, W2, b1r)
    q = _agg_kernel(h2, src2, dst2)
    return _tc3(q[0, :N], q[1, :N], h2, d0, d1, b2r)


# speedup vs baseline: 10.3532x; 1.1463x over previous
import functools

import jax
import jax.numpy as jnp
from jax import lax
from jax.experimental import pallas as pl
from jax.experimental.pallas import tpu as pltpu
from jax.experimental.pallas import tpu_sc as plsc

N = 10000
E = 320000
D = 128

NC = 2
NS = 16
CHUNK = 128
E_PAD = 327680
CHUNKS_TOTAL = E_PAD // CHUNK
CHUNKS_PER_SC = CHUNKS_TOTAL // NC
CHUNKS_PER_TILE = CHUNKS_PER_SC // NS
ACC_ROWS = 10240
ROWS_PER_TILE = ACC_ROWS // NS
IDX_BLK = 8

_mesh = plsc.VectorSubcoreMesh(core_axis_name="c", subcore_axis_name="s")


@functools.partial(
    pl.kernel,
    out_type=jax.ShapeDtypeStruct((NC, ACC_ROWS), jnp.float32),
    mesh=_mesh,
    scratch_types=[
        pltpu.VMEM((CHUNKS_PER_TILE, CHUNK), jnp.int32),
        pltpu.VMEM((CHUNK,), jnp.float32),
        pltpu.VMEM((ROWS_PER_TILE,), jnp.float32),
        pltpu.VMEM_SHARED((ACC_ROWS,), jnp.float32),
    ],
)
def _deg_kernel(dst_hbm, out_hbm, dst_v, ones_v, zero_v, acc_s):
    c = lax.axis_index("c")
    s = lax.axis_index("s")

    def _fill(i, _):
        ones_v[pl.ds(i * 16, 16)] = jnp.full((16,), 1.0, jnp.float32)
        return ()
    lax.fori_loop(0, CHUNK // 16, _fill, (), unroll=True)

    def _zfill(i, _):
        zero_v[pl.ds(i * 16, 16)] = jnp.zeros((16,), jnp.float32)
        return ()
    lax.fori_loop(0, ROWS_PER_TILE // 16, _zfill, ())

    pltpu.sync_copy(zero_v, acc_s.at[pl.ds(s * ROWS_PER_TILE, ROWS_PER_TILE)])
    plsc.subcore_barrier()

    chunk0 = c * CHUNKS_PER_SC + s * CHUNKS_PER_TILE
    pltpu.sync_copy(dst_hbm.at[pl.ds(chunk0, CHUNKS_PER_TILE)], dst_v)

    def _body(j, _):
        pltpu.sync_copy(ones_v, acc_s.at[dst_v.at[j]], add=True)
        return ()
    lax.fori_loop(0, CHUNKS_PER_TILE, _body, ())

    plsc.subcore_barrier()
    pltpu.sync_copy(
        acc_s.at[pl.ds(s * ROWS_PER_TILE, ROWS_PER_TILE)],
        out_hbm.at[c, pl.ds(s * ROWS_PER_TILE, ROWS_PER_TILE)],
    )


@functools.partial(
    pl.kernel,
    out_type=jax.ShapeDtypeStruct((NC, ACC_ROWS, D), jnp.float32),
    mesh=_mesh,
    scratch_types=[
        pltpu.VMEM((2, IDX_BLK, CHUNK), jnp.int32),
        pltpu.VMEM((2, IDX_BLK, CHUNK), jnp.int32),
        pltpu.VMEM((CHUNK, D), jnp.float32),
        pltpu.VMEM((CHUNK, D), jnp.float32),
        pltpu.VMEM((64, D), jnp.float32),
        pltpu.VMEM_SHARED((ACC_ROWS, D), jnp.float32),
        pltpu.SemaphoreType.DMA,
        pltpu.SemaphoreType.DMA,
        pltpu.SemaphoreType.DMA,
        pltpu.SemaphoreType.DMA,
    ],
)
def _agg_kernel(h_hbm, src_hbm, dst_hbm, out_hbm,
                src_v, dst_v, rows_a, rows_b, zero_v, acc_s,
                sem_ga, sem_gb, isem_a, isem_b):
    c = lax.axis_index("c")
    s = lax.axis_index("s")

    def _zfill(i, _):
        def _zl(l, _):
            zero_v[i, pl.ds(l * 16, 16)] = jnp.zeros((16,), jnp.float32)
            return ()
        lax.fori_loop(0, D // 16, _zl, (), unroll=True)
        return ()
    lax.fori_loop(0, 64, _zfill, ())

    row0 = s * ROWS_PER_TILE

    def _zcopy(j, _):
        pltpu.sync_copy(zero_v, acc_s.at[pl.ds(row0 + j * 64, 64)])
        return ()
    lax.fori_loop(0, ROWS_PER_TILE // 64, _zcopy, ())
    plsc.subcore_barrier()

    chunk0 = c * CHUNKS_PER_SC + s * CHUNKS_PER_TILE
    rows = (rows_a, rows_b)
    gsems = (sem_ga, sem_gb)
    isems = (isem_a, isem_b)
    n_blk = CHUNKS_PER_TILE // IDX_BLK

    def _idx_copy(g, wait):
        ib = g % 2
        sc = pltpu.make_async_copy(
            src_hbm.at[pl.ds(chunk0 + g * IDX_BLK, IDX_BLK)],
            src_v.at[ib], isems[ib])
        dc = pltpu.make_async_copy(
            dst_hbm.at[pl.ds(chunk0 + g * IDX_BLK, IDX_BLK)],
            dst_v.at[ib], isems[ib])
        if wait:
            sc.wait()
            dc.wait()
        else:
            sc.start()
            dc.start()

    def _gather(k):
        g, j = divmod(k, IDX_BLK)
        pltpu.async_copy(h_hbm.at[src_v.at[g % 2, j]], rows[k % 2],
                         gsems[k % 2])

    pltpu.sync_copy(src_hbm.at[pl.ds(chunk0, IDX_BLK)], src_v.at[0])
    pltpu.sync_copy(dst_hbm.at[pl.ds(chunk0, IDX_BLK)], dst_v.at[0])
    _gather(0)
    for k in range(CHUNKS_PER_TILE):
        g, j = divmod(k, IDX_BLK)
        if k + 1 < CHUNKS_PER_TILE:
            g1, j1 = divmod(k + 1, IDX_BLK)
            if j1 == 0:
                _idx_copy(g1, wait=True)
            _gather(k + 1)
        pltpu.make_async_copy(h_hbm.at[src_v.at[g % 2, j]], rows[k % 2],
                              gsems[k % 2]).wait()
        if j == 0 and g + 1 < n_blk:
            _idx_copy(g + 1, wait=False)
        pltpu.sync_copy(rows[k % 2], acc_s.at[dst_v.at[g % 2, j]], add=True)

    plsc.subcore_barrier()
    pltpu.sync_copy(
        acc_s.at[pl.ds(row0, ROWS_PER_TILE)],
        out_hbm.at[c, pl.ds(row0, ROWS_PER_TILE)],
    )


_ROW_BLK = 2000
_row_spec = pl.BlockSpec((_ROW_BLK, D), lambda i: (i, 0))
_deg_spec = pl.BlockSpec((_ROW_BLK, 1), lambda i: (i, 0))
_w_spec = pl.BlockSpec((D, D), lambda i: (0, 0))
_b_spec = pl.BlockSpec((1, D), lambda i: (0, 0))


def _tc1_body(x_ref, w_ref, d0_ref, d1_ref, o_ref):
    dis = lax.rsqrt(d0_ref[...] + d1_ref[...] + 1.0)
    h = jnp.dot(x_ref[...], w_ref[...], preferred_element_type=jnp.float32)
    o_ref[...] = h * dis


def _tc2_body(p0_ref, p1_ref, h_ref, d0_ref, d1_ref, w_ref, b_ref, o_ref):
    dis = lax.rsqrt(d0_ref[...] + d1_ref[...] + 1.0)
    z = (p0_ref[...] + p1_ref[...] + h_ref[...]) * dis + b_ref[...]
    r = jnp.maximum(z, 0.0)
    o_ref[...] = jnp.dot(r, w_ref[...], preferred_element_type=jnp.float32) * dis


def _tc3_body(q0_ref, q1_ref, h_ref, d0_ref, d1_ref, b_ref, o_ref):
    dis = lax.rsqrt(d0_ref[...] + d1_ref[...] + 1.0)
    o_ref[...] = (q0_ref[...] + q1_ref[...] + h_ref[...]) * dis + b_ref[...]


_out_nd = jax.ShapeDtypeStruct((N, D), jnp.float32)

_tc1 = pl.pallas_call(
    _tc1_body, grid=(N // _ROW_BLK,),
    in_specs=[_row_spec, _w_spec, _deg_spec, _deg_spec],
    out_specs=_row_spec, out_shape=_out_nd)

_tc2 = pl.pallas_call(
    _tc2_body, grid=(N // _ROW_BLK,),
    in_specs=[_row_spec, _row_spec, _row_spec, _deg_spec, _deg_spec,
              _w_spec, _b_spec],
    out_specs=_row_spec, out_shape=_out_nd)

_tc3 = pl.pallas_call(
    _tc3_body, grid=(N // _ROW_BLK,),
    in_specs=[_row_spec, _row_spec, _row_spec, _deg_spec, _deg_spec, _b_spec],
    out_specs=_row_spec, out_shape=_out_nd)


@jax.jit
def kernel(x, edge_index, W1, b1, W2, b2):
    src = edge_index[0].astype(jnp.int32)
    dst = edge_index[1].astype(jnp.int32)
    pad = E_PAD - E
    src = jnp.concatenate([src, jnp.zeros((pad,), jnp.int32)])
    dst = jnp.concatenate([dst, jnp.full((pad,), N, jnp.int32)])
    src2 = src.reshape(CHUNKS_TOTAL, CHUNK)
    dst2 = dst.reshape(CHUNKS_TOTAL, CHUNK)
    b1r = b1.reshape(1, D)
    b2r = b2.reshape(1, D)

    deg = _deg_kernel(dst2)
    d0 = deg[0, :N].reshape(N, 1)
    d1 = deg[1, :N].reshape(N, 1)

    h1 = _tc1(x, W1, d0, d1)
    p = _agg_kernel(h1, src2, dst2)
    h2 = _tc2(p[0, :N], p[1, :N], h1, d0, d1, W2, b1r)
    q = _agg_kernel(h2, src2, dst2)
    return _tc3(q[0, :N], q[1, :N], h2, d0, d1, b2r)


# speedup vs baseline: 32.9810x; 3.1856x over previous
import functools

import jax
import jax.numpy as jnp
from jax import lax
from jax.experimental import pallas as pl
from jax.experimental.pallas import tpu as pltpu
from jax.experimental.pallas import tpu_sc as plsc

N = 10000
E = 320000
D = 128

NC = 2
NS = 16
CHUNK = 128
E_PAD = 327680
CHUNKS_TOTAL = E_PAD // CHUNK
CHUNKS_PER_SC = CHUNKS_TOTAL // NC
CHUNKS_PER_TILE = CHUNKS_PER_SC // NS
ACC_ROWS = 10240
ROWS_PER_TILE = ACC_ROWS // NS
IDX_BLK = 8

_mesh = plsc.VectorSubcoreMesh(core_axis_name="c", subcore_axis_name="s")


@functools.partial(
    pl.kernel,
    out_type=jax.ShapeDtypeStruct((NC, ACC_ROWS), jnp.float32),
    mesh=_mesh,
    scratch_types=[
        pltpu.VMEM((CHUNKS_PER_TILE, CHUNK), jnp.int32),
        pltpu.VMEM((CHUNK,), jnp.float32),
        pltpu.VMEM((ROWS_PER_TILE,), jnp.float32),
        pltpu.VMEM_SHARED((ACC_ROWS,), jnp.float32),
        pltpu.SemaphoreType.DMA,
    ],
)
def _deg_kernel(dst_hbm, out_hbm, dst_v, ones_v, zero_v, acc_s, sem):
    c = lax.axis_index("c")
    s = lax.axis_index("s")

    def _fill(i, _):
        ones_v[pl.ds(i * 16, 16)] = jnp.full((16,), 1.0, jnp.float32)
        return ()
    lax.fori_loop(0, CHUNK // 16, _fill, (), unroll=True)

    def _zfill(i, _):
        zero_v[pl.ds(i * 16, 16)] = jnp.zeros((16,), jnp.float32)
        return ()
    lax.fori_loop(0, ROWS_PER_TILE // 16, _zfill, ())

    pltpu.sync_copy(zero_v, acc_s.at[pl.ds(s * ROWS_PER_TILE, ROWS_PER_TILE)])
    plsc.subcore_barrier()

    chunk0 = c * CHUNKS_PER_SC + s * CHUNKS_PER_TILE
    pltpu.sync_copy(dst_hbm.at[pl.ds(chunk0, CHUNKS_PER_TILE)], dst_v)

    def _fire(j, _):
        pltpu.async_copy(ones_v, acc_s.at[dst_v.at[j]], sem, add=True)
        return ()
    lax.fori_loop(0, CHUNKS_PER_TILE, _fire, ())

    def _drain(j, _):
        pltpu.make_async_copy(ones_v, acc_s.at[dst_v.at[j]], sem).wait()
        return ()
    lax.fori_loop(0, CHUNKS_PER_TILE, _drain, ())

    plsc.subcore_barrier()
    pltpu.sync_copy(
        acc_s.at[pl.ds(s * ROWS_PER_TILE, ROWS_PER_TILE)],
        out_hbm.at[c, pl.ds(s * ROWS_PER_TILE, ROWS_PER_TILE)],
    )


@functools.partial(
    pl.kernel,
    out_type=jax.ShapeDtypeStruct((NC, ACC_ROWS, D), jnp.float32),
    mesh=_mesh,
    scratch_types=[
        pltpu.VMEM((2, IDX_BLK, CHUNK), jnp.int32),
        pltpu.VMEM((2, IDX_BLK, CHUNK), jnp.int32),
        pltpu.VMEM((CHUNK, D), jnp.float32),
        pltpu.VMEM((CHUNK, D), jnp.float32),
        pltpu.VMEM((64, D), jnp.float32),
        pltpu.VMEM_SHARED((ACC_ROWS, D), jnp.float32),
        pltpu.SemaphoreType.DMA,
        pltpu.SemaphoreType.DMA,
        pltpu.SemaphoreType.DMA,
        pltpu.SemaphoreType.DMA,
        pltpu.SemaphoreType.DMA,
        pltpu.SemaphoreType.DMA,
    ],
)
def _agg_kernel(h_hbm, src_hbm, dst_hbm, out_hbm,
                src_v, dst_v, rows_a, rows_b, zero_v, acc_s,
                sem_ga, sem_gb, isem_a, isem_b, ssem_a, ssem_b):
    c = lax.axis_index("c")
    s = lax.axis_index("s")

    def _zfill(i, _):
        def _zl(l, _):
            zero_v[i, pl.ds(l * 16, 16)] = jnp.zeros((16,), jnp.float32)
            return ()
        lax.fori_loop(0, D // 16, _zl, (), unroll=True)
        return ()
    lax.fori_loop(0, 64, _zfill, ())

    row0 = s * ROWS_PER_TILE
    chunk0 = c * CHUNKS_PER_SC + s * CHUNKS_PER_TILE
    rows = (rows_a, rows_b)
    gsems = (sem_ga, sem_gb)
    isems = (isem_a, isem_b)
    n_blk = CHUNKS_PER_TILE // IDX_BLK

    pltpu.async_copy(src_hbm.at[pl.ds(chunk0, IDX_BLK)], src_v.at[0], isem_a)
    pltpu.async_copy(dst_hbm.at[pl.ds(chunk0, IDX_BLK)], dst_v.at[0], isem_a)
    for jz in range(ROWS_PER_TILE // 64):
        pltpu.async_copy(zero_v, acc_s.at[pl.ds(row0 + jz * 64, 64)], sem_gb)
    pltpu.make_async_copy(src_hbm.at[pl.ds(chunk0, IDX_BLK)], src_v.at[0],
                          isem_a).wait()
    pltpu.make_async_copy(dst_hbm.at[pl.ds(chunk0, IDX_BLK)], dst_v.at[0],
                          isem_a).wait()
    for jz in range(ROWS_PER_TILE // 64):
        pltpu.make_async_copy(zero_v, acc_s.at[pl.ds(row0 + jz * 64, 64)],
                              sem_gb).wait()
    plsc.subcore_barrier()

    def _idx_copy(g, wait):
        ib = g % 2
        sc = pltpu.make_async_copy(
            src_hbm.at[pl.ds(chunk0 + g * IDX_BLK, IDX_BLK)],
            src_v.at[ib], isems[ib])
        dc = pltpu.make_async_copy(
            dst_hbm.at[pl.ds(chunk0 + g * IDX_BLK, IDX_BLK)],
            dst_v.at[ib], isems[ib])
        if wait:
            sc.wait()
            dc.wait()
        else:
            sc.start()
            dc.start()

    def _gather(k, wait=False):
        g, j = divmod(k, IDX_BLK)
        cp = pltpu.make_async_copy(h_hbm.at[src_v.at[g % 2, j]], rows[k % 2],
                                   gsems[k % 2])
        if wait:
            cp.wait()
        else:
            cp.start()

    ssems = (ssem_a, ssem_b)

    def _scatter(k, wait):
        g, j = divmod(k, IDX_BLK)
        if wait:
            pltpu.make_async_copy(rows[k % 2], acc_s.at[dst_v.at[g % 2, j]],
                                  ssems[k % 2]).wait()
        else:
            pltpu.async_copy(rows[k % 2], acc_s.at[dst_v.at[g % 2, j]],
                             ssems[k % 2], add=True)

    _gather(0)
    for k in range(CHUNKS_PER_TILE):
        g, j = divmod(k, IDX_BLK)
        if k + 1 < CHUNKS_PER_TILE:
            g1, j1 = divmod(k + 1, IDX_BLK)
            if j1 == 0:
                _idx_copy(g1, wait=True)
            if k >= 1:
                _scatter(k - 1, wait=True)
            _gather(k + 1)
        _gather(k, wait=True)
        if j == 0 and g + 1 < n_blk:
            _idx_copy(g + 1, wait=False)
        _scatter(k, wait=False)
    _scatter(CHUNKS_PER_TILE - 2, wait=True)
    _scatter(CHUNKS_PER_TILE - 1, wait=True)

    plsc.subcore_barrier()
    pltpu.sync_copy(
        acc_s.at[pl.ds(row0, ROWS_PER_TILE)],
        out_hbm.at[c, pl.ds(row0, ROWS_PER_TILE)],
    )


_ROW_BLK = 2000
_row_spec = pl.BlockSpec((_ROW_BLK, D), lambda i: (i, 0))
_deg_spec = pl.BlockSpec((_ROW_BLK, 1), lambda i: (i, 0))
_w_spec = pl.BlockSpec((D, D), lambda i: (0, 0))
_b_spec = pl.BlockSpec((1, D), lambda i: (0, 0))


def _tc1_body(x_ref, w_ref, d0_ref, d1_ref, o_ref):
    dis = lax.rsqrt(d0_ref[...] + d1_ref[...] + 1.0)
    h = jnp.dot(x_ref[...], w_ref[...], preferred_element_type=jnp.float32)
    o_ref[...] = h * dis


def _tc2_body(p0_ref, p1_ref, h_ref, d0_ref, d1_ref, w_ref, b_ref, o_ref):
    dis = lax.rsqrt(d0_ref[...] + d1_ref[...] + 1.0)
    z = (p0_ref[...] + p1_ref[...] + h_ref[...]) * dis + b_ref[...]
    r = jnp.maximum(z, 0.0)
    o_ref[...] = jnp.dot(r, w_ref[...], preferred_element_type=jnp.float32) * dis


def _tc3_body(q0_ref, q1_ref, h_ref, d0_ref, d1_ref, b_ref, o_ref):
    dis = lax.rsqrt(d0_ref[...] + d1_ref[...] + 1.0)
    o_ref[...] = (q0_ref[...] + q1_ref[...] + h_ref[...]) * dis + b_ref[...]


_out_nd = jax.ShapeDtypeStruct((N, D), jnp.float32)

_tc1 = pl.pallas_call(
    _tc1_body, grid=(N // _ROW_BLK,),
    in_specs=[_row_spec, _w_spec, _deg_spec, _deg_spec],
    out_specs=_row_spec, out_shape=_out_nd)

_tc2 = pl.pallas_call(
    _tc2_body, grid=(N // _ROW_BLK,),
    in_specs=[_row_spec, _row_spec, _row_spec, _deg_spec, _deg_spec,
              _w_spec, _b_spec],
    out_specs=_row_spec, out_shape=_out_nd)

_tc3 = pl.pallas_call(
    _tc3_body, grid=(N // _ROW_BLK,),
    in_specs=[_row_spec, _row_spec, _row_spec, _deg_spec, _deg_spec, _b_spec],
    out_specs=_row_spec, out_shape=_out_nd)


@jax.jit
def kernel(x, edge_index, W1, b1, W2, b2):
    src = edge_index[0].astype(jnp.int32)
    dst = edge_index[1].astype(jnp.int32)
    pad = E_PAD - E
    pad_ids = lax.iota(jnp.int32, pad)
    src = jnp.concatenate([src, pad_ids % N])
    dst = jnp.concatenate([dst, N + pad_ids % (ACC_ROWS - N)])
    src2 = src.reshape(CHUNKS_TOTAL, CHUNK)
    dst2 = dst.reshape(CHUNKS_TOTAL, CHUNK)
    b1r = b1.reshape(1, D)
    b2r = b2.reshape(1, D)

    deg = _deg_kernel(dst2)
    d0 = deg[0, :N].reshape(N, 1)
    d1 = deg[1, :N].reshape(N, 1)

    h1 = _tc1(x, W1, d0, d1)
    p = _agg_kernel(h1, src2, dst2)
    h2 = _tc2(p[0, :N], p[1, :N], h1, d0, d1, W2, b1r)
    q = _agg_kernel(h2, src2, dst2)
    return _tc3(q[0, :N], q[1, :N], h2, d0, d1, b2r)
